# baseline (speedup 1.0000x reference)
import functools

import jax
import jax.numpy as jnp
from jax import lax
from jax.experimental import pallas as pl
from jax.experimental.pallas import tpu as pltpu
from jax.experimental.pallas import tpu_sc as plsc

N_NODES = 10000
N_EDGES = 320000
N_GRAPHS = 128
D = 146
DP = 160
EPS = 1e-5
NC = 2
NS = 16
NW = NC * NS
EDGES_PER_W = N_EDGES // NW
CHUNK = 80
NCHUNK = EDGES_PER_W // CHUNK
ROWS_PER_TILE = N_NODES // NS
NBLK = 10
BLK = N_NODES // NBLK
N_CLASSES_OUT = 10


def _sc_mesh():
    return plsc.VectorSubcoreMesh(core_axis_name="c", subcore_axis_name="s")


def _sc_params():
    return pltpu.CompilerParams(needs_layout_passes=False,
                                use_tc_tiling_on_sc=False)


N_PAD = 10240
E_TILE = N_EDGES // NS
NSLICE = N_PAD // NS


def _sc_degrees(src, dst):
    @functools.partial(
        pl.kernel,
        mesh=_sc_mesh(),
        compiler_params=_sc_params(),
        out_type=jax.ShapeDtypeStruct((2, N_PAD), jnp.float32),
        scratch_types=[
            pltpu.VMEM_SHARED((NS, 2, N_PAD), jnp.float32),
            pltpu.VMEM((N_PAD,), jnp.float32),
            pltpu.VMEM((N_PAD,), jnp.float32),
            pltpu.VMEM((E_TILE,), jnp.int32),
            pltpu.VMEM((E_TILE,), jnp.int32),
            pltpu.VMEM((NS, 2, NSLICE), jnp.float32),
            pltpu.VMEM((2, NSLICE), jnp.float32),
        ],
    )
    def k(src_hbm, dst_hbm, out_hbm, parts, sdeg_v, ddeg_v, src_v, dst_v,
          red_v, res_v):
        c = lax.axis_index("c")
        s = lax.axis_index("s")

        @pl.when(c == 0)
        def _():
            zeros = jnp.zeros((16,), jnp.float32)

            def zero_body(i, carry):
                sdeg_v[pl.ds(i * 16, 16)] = zeros
                ddeg_v[pl.ds(i * 16, 16)] = zeros
                return carry

            lax.fori_loop(0, N_PAD // 16, zero_body, 0)

            base = s * E_TILE
            pltpu.sync_copy(src_hbm.at[pl.ds(base, E_TILE)], src_v)
            pltpu.sync_copy(dst_hbm.at[pl.ds(base, E_TILE)], dst_v)

            ones = jnp.ones((16,), jnp.float32)

            def body(i, carry):
                si = src_v[pl.ds(i * 16, 16)]
                di = dst_v[pl.ds(i * 16, 16)]
                plsc.addupdate_scatter(sdeg_v, [si], ones)
                plsc.addupdate_scatter(ddeg_v, [di], ones)
                return carry

            lax.fori_loop(0, E_TILE // 16, body, 0)
            pltpu.sync_copy(sdeg_v, parts.at[s, 0])
            pltpu.sync_copy(ddeg_v, parts.at[s, 1])
            plsc.subcore_barrier()
            pltpu.sync_copy(parts.at[:, :, pl.ds(s * NSLICE, NSLICE)], red_v)

            def red_body(j, carry):
                for h in range(2):
                    acc = red_v[0, h, pl.ds(j * 16, 16)]
                    for t in range(1, NS):
                        acc = acc + red_v[t, h, pl.ds(j * 16, 16)]
                    res_v[h, pl.ds(j * 16, 16)] = acc
                return carry

            lax.fori_loop(0, NSLICE // 16, red_body, 0)
            pltpu.sync_copy(res_v.at[0], out_hbm.at[0, pl.ds(s * NSLICE, NSLICE)])
            pltpu.sync_copy(res_v.at[1], out_hbm.at[1, pl.ds(s * NSLICE, NSLICE)])

    return k(src, dst)


def _sc_agg(h_scaled, src, dst, zeros_tile):
    @functools.partial(
        pl.kernel,
        mesh=_sc_mesh(),
        compiler_params=_sc_params(),
        out_type=jax.ShapeDtypeStruct((NC, N_NODES, DP), jnp.float32),
        scratch_types=[
            pltpu.VMEM_SHARED((N_NODES, DP), jnp.float32),
            [pltpu.VMEM((CHUNK,), jnp.int32)] * 2,
            [pltpu.VMEM((CHUNK,), jnp.int32)] * 2,
            [pltpu.VMEM((CHUNK, DP), jnp.float32)] * 2,
            [pltpu.SemaphoreType.DMA] * 2,
            [pltpu.SemaphoreType.DMA] * 2,
        ],
    )
    def k(h_hbm, src_hbm, dst_hbm, z_hbm, out_hbm, acc, sidx, didx,
          rows, gsem, tsem):
        c = lax.axis_index("c")
        s = lax.axis_index("s")
        w = c * NS + s
        ebase = w * EDGES_PER_W

        def load_idx(b, chunk):
            pltpu.sync_copy(src_hbm.at[pl.ds(ebase + chunk * CHUNK, CHUNK)],
                            sidx[b])
            pltpu.sync_copy(dst_hbm.at[pl.ds(ebase + chunk * CHUNK, CHUNK)],
                            didx[b])

        def gather(b):
            pltpu.async_copy(h_hbm.at[sidx[b]], rows[b], gsem[b])

        def wait_gather(b):
            pltpu.make_async_copy(h_hbm.at[sidx[b]], rows[b], gsem[b]).wait()

        def scatter(b):
            pltpu.async_copy(rows[b], acc.at[didx[b]], tsem[b], add=True)

        def wait_scatter(b):
            pltpu.make_async_copy(rows[b], acc.at[didx[b]], tsem[b]).wait()

        load_idx(0, 0)
        gather(0)
        load_idx(1, 1)
        gather(1)
        pltpu.sync_copy(z_hbm, acc.at[pl.ds(s * ROWS_PER_TILE, ROWS_PER_TILE)])
        plsc.subcore_barrier()

        def body(j, carry):
            wait_gather(0)
            scatter(0)
            wait_gather(1)
            scatter(1)
            wait_scatter(0)
            load_idx(0, 2 * j + 2)
            gather(0)
            wait_scatter(1)

            @pl.when(2 * j + 3 < NCHUNK)
            def _():
                load_idx(1, 2 * j + 3)
                gather(1)

            return carry

        lax.fori_loop(0, (NCHUNK - 1) // 2, body, 0)
        wait_gather(0)
        scatter(0)
        wait_scatter(0)
        plsc.subcore_barrier()
        pltpu.sync_copy(
            acc.at[pl.ds(s * ROWS_PER_TILE, ROWS_PER_TILE)],
            out_hbm.at[c, pl.ds(s * ROWS_PER_TILE, ROWS_PER_TILE)],
        )

    return k(h_scaled, src, dst, zeros_tile)


def _norms_kernel(deg2):
    def body(d_ref, out_ref):
        out_ref[...] = lax.rsqrt(jnp.clip(d_ref[...], 1.0, None))

    return pl.pallas_call(
        body,
        grid=(NBLK,),
        in_specs=[pl.BlockSpec((2, N_PAD // NBLK), lambda i: (0, i))],
        out_specs=pl.BlockSpec((2, N_PAD // NBLK), lambda i: (0, i)),
        out_shape=jax.ShapeDtypeStruct((2, N_PAD), jnp.float32),
    )(deg2)


def _embed_kernel(xp, wp, bp, norms):
    def body(x_ref, w_ref, b_ref, n_ref, h_ref, hs_ref):
        h = jnp.dot(x_ref[...], w_ref[...], preferred_element_type=jnp.float32)
        h = h + b_ref[...]
        h_ref[...] = h
        hs_ref[...] = h * n_ref[:, 0:1]

    return pl.pallas_call(
        body,
        grid=(NBLK,),
        in_specs=[
            pl.BlockSpec((BLK, DP), lambda i: (i, 0)),
            pl.BlockSpec((DP, DP), lambda i: (0, 0)),
            pl.BlockSpec((1, DP), lambda i: (0, 0)),
            pl.BlockSpec((BLK, 2), lambda i: (i, 0)),
        ],
        out_specs=[
            pl.BlockSpec((BLK, DP), lambda i: (i, 0)),
            pl.BlockSpec((BLK, DP), lambda i: (i, 0)),
        ],
        out_shape=[
            jax.ShapeDtypeStruct((N_NODES, DP), jnp.float32),
            jax.ShapeDtypeStruct((N_NODES, DP), jnp.float32),
        ],
    )(xp, wp, bp, norms)


def _layer_mm_kernel(parts, norms, snorm, wp, bp):
    def body(p_ref, n_ref, sn_ref, w_ref, b_ref, z_ref, st_ref):
        i = pl.program_id(0)
        agg = (p_ref[0] + p_ref[1]) * n_ref[:, 1:2]
        z = jnp.dot(agg, w_ref[...], preferred_element_type=jnp.float32)
        z = (z + b_ref[...]) * sn_ref[...]
        z_ref[...] = z

        @pl.when(i == 0)
        def _():
            st_ref[...] = jnp.zeros_like(st_ref)

        st_ref[0:1, :] += jnp.sum(z, axis=0, keepdims=True)
        st_ref[1:2, :] += jnp.sum(z * z, axis=0, keepdims=True)

    return pl.pallas_call(
        body,
        grid=(NBLK,),
        in_specs=[
            pl.BlockSpec((NC, BLK, DP), lambda i: (0, i, 0)),
            pl.BlockSpec((BLK, 2), lambda i: (i, 0)),
            pl.BlockSpec((BLK, 1), lambda i: (i, 0)),
            pl.BlockSpec((DP, DP), lambda i: (0, 0)),
            pl.BlockSpec((1, DP), lambda i: (0, 0)),
        ],
        out_specs=[
            pl.BlockSpec((BLK, DP), lambda i: (i, 0)),
            pl.BlockSpec((8, DP), lambda i: (0, 0)),
        ],
        out_shape=[
            jax.ShapeDtypeStruct((N_NODES, DP), jnp.float32),
            jax.ShapeDtypeStruct((8, DP), jnp.float32),
        ],
    )(parts, norms, snorm, wp, bp)


def _layer_bn_kernel(z, stats, h_in, norms, gp, bep):
    def body(z_ref, st_ref, h_ref, n_ref, g_ref, be_ref, o_ref, os_ref):
        inv_n = 1.0 / N_NODES
        mean = st_ref[0:1, :] * inv_n
        var = st_ref[1:2, :] * inv_n - mean * mean
        hn = (z_ref[...] - mean) * lax.rsqrt(var + EPS)
        hn = hn * g_ref[...] + be_ref[...]
        h = h_ref[...] + jnp.maximum(hn, 0.0)
        o_ref[...] = h
        os_ref[...] = h * n_ref[:, 0:1]

    return pl.pallas_call(
        body,
        grid=(NBLK,),
        in_specs=[
            pl.BlockSpec((BLK, DP), lambda i: (i, 0)),
            pl.BlockSpec((8, DP), lambda i: (0, 0)),
            pl.BlockSpec((BLK, DP), lambda i: (i, 0)),
            pl.BlockSpec((BLK, 2), lambda i: (i, 0)),
            pl.BlockSpec((1, DP), lambda i: (0, 0)),
            pl.BlockSpec((1, DP), lambda i: (0, 0)),
        ],
        out_specs=[
            pl.BlockSpec((BLK, DP), lambda i: (i, 0)),
            pl.BlockSpec((BLK, DP), lambda i: (i, 0)),
        ],
        out_shape=[
            jax.ShapeDtypeStruct((N_NODES, DP), jnp.float32),
            jax.ShapeDtypeStruct((N_NODES, DP), jnp.float32),
        ],
    )(z, stats, h_in, norms, gp, bep)


def _pool_mlp_kernel(h, gid2d, wr0, br0, wr1, br1, wr2, br2):
    def body(h_ref, g_ref, w0_ref, b0_ref, w1_ref, b1_ref, w2_ref, b2_ref,
             o_ref, acc, cnt):
        i = pl.program_id(0)

        @pl.when(i == 0)
        def _():
            acc[...] = jnp.zeros_like(acc)
            cnt[...] = jnp.zeros_like(cnt)

        giota = lax.broadcasted_iota(jnp.int32, (BLK, N_GRAPHS), 1)
        onehot = (g_ref[...] == giota).astype(jnp.float32)
        acc[...] += lax.dot_general(
            onehot, h_ref[...], (((0,), (0,)), ((), ())),
            preferred_element_type=jnp.float32)
        cnt[...] += lax.dot_general(
            onehot, jnp.ones((BLK, 8), jnp.float32), (((0,), (0,)), ((), ())),
            preferred_element_type=jnp.float32)

        @pl.when(i == NBLK - 1)
        def _():
            hg = acc[...] / jnp.clip(cnt[:, 0:1], 1.0, None)
            y = jnp.dot(hg, w0_ref[...], preferred_element_type=jnp.float32)
            y = jnp.maximum(y + b0_ref[...], 0.0)
            y = jnp.dot(y, w1_ref[...], preferred_element_type=jnp.float32)
            y = jnp.maximum(y + b1_ref[...], 0.0)
            y = jnp.dot(y, w2_ref[...], preferred_element_type=jnp.float32)
            o_ref[...] = y + b2_ref[...]

    return pl.pallas_call(
        body,
        grid=(NBLK,),
        in_specs=[
            pl.BlockSpec((BLK, DP), lambda i: (i, 0)),
            pl.BlockSpec((BLK, 1), lambda i: (i, 0)),
            pl.BlockSpec((DP, 80), lambda i: (0, 0)),
            pl.BlockSpec((1, 80), lambda i: (0, 0)),
            pl.BlockSpec((80, 48), lambda i: (0, 0)),
            pl.BlockSpec((1, 48), lambda i: (0, 0)),
            pl.BlockSpec((48, 128), lambda i: (0, 0)),
            pl.BlockSpec((1, 128), lambda i: (0, 0)),
        ],
        out_specs=pl.BlockSpec((N_GRAPHS, 128), lambda i: (0, 0)),
        out_shape=jax.ShapeDtypeStruct((N_GRAPHS, 128), jnp.float32),
        scratch_shapes=[
            pltpu.VMEM((N_GRAPHS, DP), jnp.float32),
            pltpu.VMEM((N_GRAPHS, 8), jnp.float32),
        ],
    )(h, gid2d, wr0, br0, wr1, br1, wr2, br2)


def _pad2(a, r, c):
    return jnp.pad(a, ((0, r - a.shape[0]), (0, c - a.shape[1])))


def _pad_row(v, c):
    return jnp.pad(v, (0, c - v.shape[0])).reshape(1, c)


def kernel(nodes_feat, edge_index, edges_feat, nodes_num_norm_sqrt,
           edges_num_norm_sqrt, graph_ids,
           W_emb, b_emb, W1, b1, g1, be1, W2, b2, g2, be2,
           W3, b3, g3, be3, W4, b4, g4, be4,
           Wr0, br0, Wr1, br1, Wr2, br2):
    del edges_feat, edges_num_norm_sqrt

    xp = _pad2(nodes_feat, N_NODES, DP)
    zeros_tile = jnp.zeros((ROWS_PER_TILE, DP), jnp.float32)
    gid2d = graph_ids.reshape(N_NODES, 1)
    src = edge_index[0]
    dst = edge_index[1]

    deg2 = _sc_degrees(src, dst)
    norms = _norms_kernel(deg2).T[:N_NODES]

    h, hs = _embed_kernel(xp, _pad2(W_emb, DP, DP), _pad_row(b_emb, DP), norms)

    layer_params = [
        (W1, b1, g1, be1), (W2, b2, g2, be2), (W3, b3, g3, be3), (W4, b4, g4, be4),
    ]
    for (W, b, g, be) in layer_params:
        parts = _sc_agg(hs, src, dst, zeros_tile)
        z, stats = _layer_mm_kernel(parts, norms, nodes_num_norm_sqrt,
                                    _pad2(W, DP, DP), _pad_row(b, DP))
        h, hs = _layer_bn_kernel(z, stats, h, norms, _pad_row(g, DP),
                                 _pad_row(be, DP))

    logits = _pool_mlp_kernel(
        h, gid2d,
        _pad2(Wr0, DP, 80), _pad_row(br0, 80),
        _pad2(Wr1, 80, 48), _pad_row(br1, 48),
        _pad2(Wr2, 48, 128), _pad_row(br2, 128),
    )
    return logits[:, :N_CLASSES_OUT]


# speedup vs baseline: 1.0533x; 1.0533x over previous
import functools

import jax
import jax.numpy as jnp
from jax import lax
from jax.experimental import pallas as pl
from jax.experimental.pallas import tpu as pltpu
from jax.experimental.pallas import tpu_sc as plsc

N_NODES = 10000
N_EDGES = 320000
N_GRAPHS = 128
D = 146
DP = 160
EPS = 1e-5
NC = 2
NS = 16
NW = NC * NS
EDGES_PER_W = N_EDGES // NW
CHUNK = 80
NCHUNK = EDGES_PER_W // CHUNK
ROWS_PER_TILE = N_NODES // NS
NBLK = 10
BLK = N_NODES // NBLK
N_CLASSES_OUT = 10


def _sc_mesh():
    return plsc.VectorSubcoreMesh(core_axis_name="c", subcore_axis_name="s")


def _sc_params():
    return pltpu.CompilerParams(needs_layout_passes=False,
                                use_tc_tiling_on_sc=False)


N_PAD = 10240
E_TILE = N_EDGES // NS
NSLICE = N_PAD // NS


def _sc_degrees(src, dst):
    @functools.partial(
        pl.kernel,
        mesh=_sc_mesh(),
        compiler_params=_sc_params(),
        out_type=jax.ShapeDtypeStruct((2, N_PAD), jnp.float32),
        scratch_types=[
            pltpu.VMEM_SHARED((NS, 2, N_PAD), jnp.float32),
            pltpu.VMEM((N_PAD,), jnp.float32),
            pltpu.VMEM((N_PAD,), jnp.float32),
            pltpu.VMEM((E_TILE,), jnp.int32),
            pltpu.VMEM((E_TILE,), jnp.int32),
            pltpu.VMEM((NS, 2, NSLICE), jnp.float32),
            pltpu.VMEM((2, NSLICE), jnp.float32),
        ],
    )
    def k(src_hbm, dst_hbm, out_hbm, parts, sdeg_v, ddeg_v, src_v, dst_v,
          red_v, res_v):
        c = lax.axis_index("c")
        s = lax.axis_index("s")

        @pl.when(c == 0)
        def _():
            zeros = jnp.zeros((16,), jnp.float32)

            def zero_body(i, carry):
                sdeg_v[pl.ds(i * 16, 16)] = zeros
                ddeg_v[pl.ds(i * 16, 16)] = zeros
                return carry

            lax.fori_loop(0, N_PAD // 16, zero_body, 0)

            base = s * E_TILE
            pltpu.sync_copy(src_hbm.at[pl.ds(base, E_TILE)], src_v)
            pltpu.sync_copy(dst_hbm.at[pl.ds(base, E_TILE)], dst_v)

            ones = jnp.ones((16,), jnp.float32)

            def body(i, carry):
                si = src_v[pl.ds(i * 16, 16)]
                di = dst_v[pl.ds(i * 16, 16)]
                plsc.addupdate_scatter(sdeg_v, [si], ones)
                plsc.addupdate_scatter(ddeg_v, [di], ones)
                return carry

            lax.fori_loop(0, E_TILE // 16, body, 0)
            pltpu.sync_copy(sdeg_v, parts.at[s, 0])
            pltpu.sync_copy(ddeg_v, parts.at[s, 1])
            plsc.subcore_barrier()
            pltpu.sync_copy(parts.at[:, :, pl.ds(s * NSLICE, NSLICE)], red_v)

            def red_body(j, carry):
                for h in range(2):
                    acc = red_v[0, h, pl.ds(j * 16, 16)]
                    for t in range(1, NS):
                        acc = acc + red_v[t, h, pl.ds(j * 16, 16)]
                    res_v[h, pl.ds(j * 16, 16)] = acc
                return carry

            lax.fori_loop(0, NSLICE // 16, red_body, 0)
            pltpu.sync_copy(res_v.at[0], out_hbm.at[0, pl.ds(s * NSLICE, NSLICE)])
            pltpu.sync_copy(res_v.at[1], out_hbm.at[1, pl.ds(s * NSLICE, NSLICE)])

    return k(src, dst)


def _sc_agg(h_scaled, src, dst, zeros_tile):
    @functools.partial(
        pl.kernel,
        mesh=_sc_mesh(),
        compiler_params=_sc_params(),
        out_type=jax.ShapeDtypeStruct((NC, N_NODES, DP), jnp.float32),
        scratch_types=[
            pltpu.VMEM_SHARED((N_NODES, DP), jnp.float32),
            [pltpu.VMEM((CHUNK,), jnp.int32)] * 2,
            [pltpu.VMEM((CHUNK,), jnp.int32)] * 2,
            [pltpu.VMEM((CHUNK, DP), jnp.float32)] * 2,
            [pltpu.SemaphoreType.DMA] * 2,
            [pltpu.SemaphoreType.DMA] * 2,
        ],
    )
    def k(h_hbm, src_hbm, dst_hbm, z_hbm, out_hbm, acc, sidx, didx,
          rows, gsem, tsem):
        c = lax.axis_index("c")
        s = lax.axis_index("s")
        w = c * NS + s
        ebase = w * EDGES_PER_W

        def load_idx(b, chunk):
            pltpu.sync_copy(src_hbm.at[pl.ds(ebase + chunk * CHUNK, CHUNK)],
                            sidx[b])
            pltpu.sync_copy(dst_hbm.at[pl.ds(ebase + chunk * CHUNK, CHUNK)],
                            didx[b])

        def gather(b):
            pltpu.async_copy(h_hbm.at[sidx[b]], rows[b], gsem[b])

        def wait_gather(b):
            pltpu.make_async_copy(h_hbm.at[sidx[b]], rows[b], gsem[b]).wait()

        def scatter(b):
            pltpu.async_copy(rows[b], acc.at[didx[b]], tsem[b], add=True)

        def wait_scatter(b):
            pltpu.make_async_copy(rows[b], acc.at[didx[b]], tsem[b]).wait()

        load_idx(0, 0)
        gather(0)
        pltpu.sync_copy(z_hbm, acc.at[pl.ds(s * ROWS_PER_TILE, ROWS_PER_TILE)])
        plsc.subcore_barrier()

        def body(j, carry):
            @pl.when(j > 0)
            def _():
                wait_scatter(1)

            load_idx(1, 2 * j + 1)
            gather(1)
            wait_gather(0)
            scatter(0)
            wait_scatter(0)
            load_idx(0, 2 * j + 2)
            gather(0)
            wait_gather(1)
            scatter(1)
            return carry

        lax.fori_loop(0, (NCHUNK - 1) // 2, body, 0)
        wait_scatter(1)
        wait_gather(0)
        scatter(0)
        wait_scatter(0)
        plsc.subcore_barrier()
        pltpu.sync_copy(
            acc.at[pl.ds(s * ROWS_PER_TILE, ROWS_PER_TILE)],
            out_hbm.at[c, pl.ds(s * ROWS_PER_TILE, ROWS_PER_TILE)],
        )

    return k(h_scaled, src, dst, zeros_tile)


def _norms_kernel(deg2):
    NB = N_PAD // NBLK

    def body(d_ref, out_ref):
        nrm = lax.rsqrt(jnp.clip(d_ref[...], 1.0, None))
        rows = lax.broadcasted_iota(jnp.int32, (256, 256), 0)
        cols = lax.broadcasted_iota(jnp.int32, (256, 256), 1)
        eye = (rows == cols).astype(jnp.float32)
        for t in range(NB // 256):
            blk = nrm[:, t * 256:(t + 1) * 256]
            out_ref[t * 256:(t + 1) * 256, :] = lax.dot_general(
                eye, blk, (((1,), (1,)), ((), ())),
                preferred_element_type=jnp.float32)

    return pl.pallas_call(
        body,
        grid=(NBLK,),
        in_specs=[pl.BlockSpec((2, NB), lambda i: (0, i))],
        out_specs=pl.BlockSpec((NB, 2), lambda i: (i, 0)),
        out_shape=jax.ShapeDtypeStruct((N_PAD, 2), jnp.float32),
    )(deg2)


def _embed_kernel(xp, wp, bp, norms):
    def body(x_ref, w_ref, b_ref, n_ref, h_ref, hs_ref):
        h = jnp.dot(x_ref[...], w_ref[...], preferred_element_type=jnp.float32)
        h = h + b_ref[...]
        h_ref[...] = h
        hs_ref[...] = h * n_ref[:, 0:1]

    return pl.pallas_call(
        body,
        grid=(NBLK,),
        in_specs=[
            pl.BlockSpec((BLK, DP), lambda i: (i, 0)),
            pl.BlockSpec((DP, DP), lambda i: (0, 0)),
            pl.BlockSpec((1, DP), lambda i: (0, 0)),
            pl.BlockSpec((BLK, 2), lambda i: (i, 0)),
        ],
        out_specs=[
            pl.BlockSpec((BLK, DP), lambda i: (i, 0)),
            pl.BlockSpec((BLK, DP), lambda i: (i, 0)),
        ],
        out_shape=[
            jax.ShapeDtypeStruct((N_NODES, DP), jnp.float32),
            jax.ShapeDtypeStruct((N_NODES, DP), jnp.float32),
        ],
    )(xp, wp, bp, norms)


def _layer_kernel(parts, norms, snorm, h_in, wp, bp, gp, bep):
    def body(p_ref, n_ref, sn_ref, h_ref, w_ref, b_ref, g_ref, be_ref,
             o_ref, os_ref, z_sc, st_sc):
        j = pl.program_id(0)
        i = pl.program_id(1)

        @pl.when(j == 0)
        def _():
            @pl.when(i == 0)
            def _():
                st_sc[...] = jnp.zeros_like(st_sc)

            agg = (p_ref[0] + p_ref[1]) * n_ref[:, 1:2]
            z = jnp.dot(agg, w_ref[...], preferred_element_type=jnp.float32)
            z = (z + b_ref[...]) * sn_ref[...]
            z_sc[pl.ds(i * BLK, BLK), :] = z
            st_sc[0:1, :] += jnp.sum(z, axis=0, keepdims=True)
            st_sc[1:2, :] += jnp.sum(z * z, axis=0, keepdims=True)

        @pl.when(j == 1)
        def _():
            inv_n = 1.0 / N_NODES
            mean = st_sc[0:1, :] * inv_n
            var = st_sc[1:2, :] * inv_n - mean * mean
            zblk = z_sc[pl.ds(i * BLK, BLK), :]
            hn = (zblk - mean) * lax.rsqrt(var + EPS)
            hn = hn * g_ref[...] + be_ref[...]
            h = h_ref[...] + jnp.maximum(hn, 0.0)
            o_ref[...] = h
            os_ref[...] = h * n_ref[:, 0:1]

    return pl.pallas_call(
        body,
        grid=(2, NBLK),
        in_specs=[
            pl.BlockSpec((NC, BLK, DP), lambda j, i: (0, i * (1 - j), 0)),
            pl.BlockSpec((BLK, 2), lambda j, i: (i, 0)),
            pl.BlockSpec((BLK, 1), lambda j, i: (i * (1 - j), 0)),
            pl.BlockSpec((BLK, DP), lambda j, i: (i * j, 0)),
            pl.BlockSpec((DP, DP), lambda j, i: (0, 0)),
            pl.BlockSpec((1, DP), lambda j, i: (0, 0)),
            pl.BlockSpec((1, DP), lambda j, i: (0, 0)),
            pl.BlockSpec((1, DP), lambda j, i: (0, 0)),
        ],
        out_specs=[
            pl.BlockSpec((BLK, DP), lambda j, i: (i * j, 0)),
            pl.BlockSpec((BLK, DP), lambda j, i: (i * j, 0)),
        ],
        out_shape=[
            jax.ShapeDtypeStruct((N_NODES, DP), jnp.float32),
            jax.ShapeDtypeStruct((N_NODES, DP), jnp.float32),
        ],
        scratch_shapes=[
            pltpu.VMEM((N_NODES, DP), jnp.float32),
            pltpu.VMEM((8, DP), jnp.float32),
        ],
    )(parts, norms, snorm, h_in, wp, bp, gp, bep)


def _pool_mlp_kernel(h, gid2d, wr0, br0, wr1, br1, wr2, br2):
    def body(h_ref, g_ref, w0_ref, b0_ref, w1_ref, b1_ref, w2_ref, b2_ref,
             o_ref, acc, cnt):
        i = pl.program_id(0)

        @pl.when(i == 0)
        def _():
            acc[...] = jnp.zeros_like(acc)
            cnt[...] = jnp.zeros_like(cnt)

        giota = lax.broadcasted_iota(jnp.int32, (BLK, N_GRAPHS), 1)
        onehot = (g_ref[...] == giota).astype(jnp.float32)
        acc[...] += lax.dot_general(
            onehot, h_ref[...], (((0,), (0,)), ((), ())),
            preferred_element_type=jnp.float32)
        cnt[...] += lax.dot_general(
            onehot, jnp.ones((BLK, 8), jnp.float32), (((0,), (0,)), ((), ())),
            preferred_element_type=jnp.float32)

        @pl.when(i == NBLK - 1)
        def _():
            hg = acc[...] / jnp.clip(cnt[:, 0:1], 1.0, None)
            y = jnp.dot(hg, w0_ref[...], preferred_element_type=jnp.float32)
            y = jnp.maximum(y + b0_ref[...], 0.0)
            y = jnp.dot(y, w1_ref[...], preferred_element_type=jnp.float32)
            y = jnp.maximum(y + b1_ref[...], 0.0)
            y = jnp.dot(y, w2_ref[...], preferred_element_type=jnp.float32)
            o_ref[...] = y + b2_ref[...]

    return pl.pallas_call(
        body,
        grid=(NBLK,),
        in_specs=[
            pl.BlockSpec((BLK, DP), lambda i: (i, 0)),
            pl.BlockSpec((BLK, 1), lambda i: (i, 0)),
            pl.BlockSpec((DP, 80), lambda i: (0, 0)),
            pl.BlockSpec((1, 80), lambda i: (0, 0)),
            pl.BlockSpec((80, 48), lambda i: (0, 0)),
            pl.BlockSpec((1, 48), lambda i: (0, 0)),
            pl.BlockSpec((48, 128), lambda i: (0, 0)),
            pl.BlockSpec((1, 128), lambda i: (0, 0)),
        ],
        out_specs=pl.BlockSpec((N_GRAPHS, 128), lambda i: (0, 0)),
        out_shape=jax.ShapeDtypeStruct((N_GRAPHS, 128), jnp.float32),
        scratch_shapes=[
            pltpu.VMEM((N_GRAPHS, DP), jnp.float32),
            pltpu.VMEM((N_GRAPHS, 8), jnp.float32),
        ],
    )(h, gid2d, wr0, br0, wr1, br1, wr2, br2)


def _pad2(a, r, c):
    return jnp.pad(a, ((0, r - a.shape[0]), (0, c - a.shape[1])))


def _pad_row(v, c):
    return jnp.pad(v, (0, c - v.shape[0])).reshape(1, c)


def kernel(nodes_feat, edge_index, edges_feat, nodes_num_norm_sqrt,
           edges_num_norm_sqrt, graph_ids,
           W_emb, b_emb, W1, b1, g1, be1, W2, b2, g2, be2,
           W3, b3, g3, be3, W4, b4, g4, be4,
           Wr0, br0, Wr1, br1, Wr2, br2):
    del edges_feat, edges_num_norm_sqrt

    xp = _pad2(nodes_feat, N_NODES, DP)
    zeros_tile = jnp.zeros((ROWS_PER_TILE, DP), jnp.float32)
    gid2d = graph_ids.reshape(N_NODES, 1)
    src = edge_index[0]
    dst = edge_index[1]

    deg2 = _sc_degrees(src, dst)
    norms = _norms_kernel(deg2)

    h, hs = _embed_kernel(xp, _pad2(W_emb, DP, DP), _pad_row(b_emb, DP), norms)

    layer_params = [
        (W1, b1, g1, be1), (W2, b2, g2, be2), (W3, b3, g3, be3), (W4, b4, g4, be4),
    ]
    for (W, b, g, be) in layer_params:
        parts = _sc_agg(hs, src, dst, zeros_tile)
        h, hs = _layer_kernel(parts, norms, nodes_num_norm_sqrt, h,
                              _pad2(W, DP, DP), _pad_row(b, DP),
                              _pad_row(g, DP), _pad_row(be, DP))

    logits = _pool_mlp_kernel(
        h, gid2d,
        _pad2(Wr0, DP, 80), _pad_row(br0, 80),
        _pad2(Wr1, 80, 48), _pad_row(br1, 48),
        _pad2(Wr2, 48, 128), _pad_row(br2, 128),
    )
    return logits[:, :N_CLASSES_OUT]


# speedup vs baseline: 1.0807x; 1.0261x over previous
import functools

import jax
import jax.numpy as jnp
from jax import lax
from jax.experimental import pallas as pl
from jax.experimental.pallas import tpu as pltpu
from jax.experimental.pallas import tpu_sc as plsc

N_NODES = 10000
N_EDGES = 320000
N_GRAPHS = 128
D = 146
DP = 160
EPS = 1e-5
NC = 2
NS = 16
NW = NC * NS
EDGES_PER_W = N_EDGES // NW
CHUNK = 80
NCHUNK = EDGES_PER_W // CHUNK
ROWS_PER_TILE = N_NODES // NS
NBLK = 10
BLK = N_NODES // NBLK
N_CLASSES_OUT = 10


def _sc_mesh():
    return plsc.VectorSubcoreMesh(core_axis_name="c", subcore_axis_name="s")


def _sc_params():
    return pltpu.CompilerParams(needs_layout_passes=False,
                                use_tc_tiling_on_sc=False)


N_PAD = 10240
E_TILE = N_EDGES // NS
NSLICE = N_PAD // NS


def _sc_degrees(src, dst):
    @functools.partial(
        pl.kernel,
        mesh=_sc_mesh(),
        compiler_params=_sc_params(),
        out_type=jax.ShapeDtypeStruct((2, N_PAD), jnp.float32),
        scratch_types=[
            pltpu.VMEM_SHARED((NS, 2, N_PAD), jnp.float32),
            pltpu.VMEM((N_PAD,), jnp.float32),
            pltpu.VMEM((N_PAD,), jnp.float32),
            pltpu.VMEM((E_TILE,), jnp.int32),
            pltpu.VMEM((E_TILE,), jnp.int32),
            pltpu.VMEM((NS, 2, NSLICE), jnp.float32),
            pltpu.VMEM((2, NSLICE), jnp.float32),
        ],
    )
    def k(src_hbm, dst_hbm, out_hbm, parts, sdeg_v, ddeg_v, src_v, dst_v,
          red_v, res_v):
        c = lax.axis_index("c")
        s = lax.axis_index("s")

        @pl.when(c == 0)
        def _():
            zeros = jnp.zeros((16,), jnp.float32)

            def zero_body(i, carry):
                sdeg_v[pl.ds(i * 16, 16)] = zeros
                ddeg_v[pl.ds(i * 16, 16)] = zeros
                return carry

            lax.fori_loop(0, N_PAD // 16, zero_body, 0)

            base = s * E_TILE
            pltpu.sync_copy(src_hbm.at[pl.ds(base, E_TILE)], src_v)
            pltpu.sync_copy(dst_hbm.at[pl.ds(base, E_TILE)], dst_v)

            ones = jnp.ones((16,), jnp.float32)

            def body(i, carry):
                si = src_v[pl.ds(i * 16, 16)]
                di = dst_v[pl.ds(i * 16, 16)]
                plsc.addupdate_scatter(sdeg_v, [si], ones)
                plsc.addupdate_scatter(ddeg_v, [di], ones)
                return carry

            lax.fori_loop(0, E_TILE // 16, body, 0)
            pltpu.sync_copy(sdeg_v, parts.at[s, 0])
            pltpu.sync_copy(ddeg_v, parts.at[s, 1])
            plsc.subcore_barrier()
            pltpu.sync_copy(parts.at[:, :, pl.ds(s * NSLICE, NSLICE)], red_v)

            def red_body(j, carry):
                for h in range(2):
                    acc = red_v[0, h, pl.ds(j * 16, 16)]
                    for t in range(1, NS):
                        acc = acc + red_v[t, h, pl.ds(j * 16, 16)]
                    res_v[h, pl.ds(j * 16, 16)] = acc
                return carry

            lax.fori_loop(0, NSLICE // 16, red_body, 0)
            pltpu.sync_copy(res_v.at[0], out_hbm.at[0, pl.ds(s * NSLICE, NSLICE)])
            pltpu.sync_copy(res_v.at[1], out_hbm.at[1, pl.ds(s * NSLICE, NSLICE)])

    return k(src, dst)


def _sc_agg(h_scaled, src, dst, zeros_tile):
    @functools.partial(
        pl.kernel,
        mesh=_sc_mesh(),
        compiler_params=_sc_params(),
        out_type=jax.ShapeDtypeStruct((NC, N_NODES, DP), jnp.float32),
        scratch_types=[
            pltpu.VMEM_SHARED((N_NODES, DP), jnp.float32),
            [pltpu.VMEM((CHUNK,), jnp.int32)] * 2,
            [pltpu.VMEM((CHUNK,), jnp.int32)] * 2,
            [pltpu.VMEM((CHUNK, DP), jnp.float32)] * 2,
            [pltpu.SemaphoreType.DMA] * 2,
            [pltpu.SemaphoreType.DMA] * 2,
        ],
    )
    def k(h_hbm, src_hbm, dst_hbm, z_hbm, out_hbm, acc, sidx, didx,
          rows, gsem, tsem):
        c = lax.axis_index("c")
        s = lax.axis_index("s")
        w = c * NS + s
        ebase = w * EDGES_PER_W

        def load_idx(b, chunk):
            pltpu.sync_copy(src_hbm.at[pl.ds(ebase + chunk * CHUNK, CHUNK)],
                            sidx[b])
            pltpu.sync_copy(dst_hbm.at[pl.ds(ebase + chunk * CHUNK, CHUNK)],
                            didx[b])

        def gather(b):
            pltpu.async_copy(h_hbm.at[sidx[b]], rows[b], gsem[b])

        def wait_gather(b):
            pltpu.make_async_copy(h_hbm.at[sidx[b]], rows[b], gsem[b]).wait()

        def scatter(b):
            pltpu.async_copy(rows[b], acc.at[didx[b]], tsem[b], add=True)

        def wait_scatter(b):
            pltpu.make_async_copy(rows[b], acc.at[didx[b]], tsem[b]).wait()

        load_idx(0, 0)
        gather(0)
        pltpu.sync_copy(z_hbm, acc.at[pl.ds(s * ROWS_PER_TILE, ROWS_PER_TILE)])
        plsc.subcore_barrier()

        def body(j, carry):
            @pl.when(j > 0)
            def _():
                wait_scatter(1)

            load_idx(1, 2 * j + 1)
            gather(1)
            wait_gather(0)
            scatter(0)
            wait_scatter(0)
            load_idx(0, 2 * j + 2)
            gather(0)
            wait_gather(1)
            scatter(1)
            return carry

        lax.fori_loop(0, (NCHUNK - 1) // 2, body, 0)
        wait_scatter(1)
        wait_gather(0)
        scatter(0)
        wait_scatter(0)
        plsc.subcore_barrier()
        pltpu.sync_copy(
            acc.at[pl.ds(s * ROWS_PER_TILE, ROWS_PER_TILE)],
            out_hbm.at[c, pl.ds(s * ROWS_PER_TILE, ROWS_PER_TILE)],
        )

    return k(h_scaled, src, dst, zeros_tile)


def _norms_kernel(deg2):
    NB = N_PAD // NBLK

    def body(d_ref, out_ref):
        nrm = lax.rsqrt(jnp.clip(d_ref[...], 1.0, None))
        rows = lax.broadcasted_iota(jnp.int32, (256, 256), 0)
        cols = lax.broadcasted_iota(jnp.int32, (256, 256), 1)
        eye = (rows == cols).astype(jnp.float32)
        for t in range(NB // 256):
            blk = nrm[:, t * 256:(t + 1) * 256]
            out_ref[t * 256:(t + 1) * 256, :] = lax.dot_general(
                eye, blk, (((1,), (1,)), ((), ())),
                preferred_element_type=jnp.float32)

    return pl.pallas_call(
        body,
        grid=(NBLK,),
        in_specs=[pl.BlockSpec((2, NB), lambda i: (0, i))],
        out_specs=pl.BlockSpec((NB, 2), lambda i: (i, 0)),
        out_shape=jax.ShapeDtypeStruct((N_PAD, 2), jnp.float32),
    )(deg2)


def _embed_kernel(xt, wp, bp, norms):
    def body(x_ref, w_ref, b_ref, n_ref, h_ref, hs_ref):
        h = lax.dot_general(x_ref[...], w_ref[...], (((0,), (0,)), ((), ())),
                            preferred_element_type=jnp.float32)
        h = h + b_ref[...]
        h_ref[...] = h
        hs_ref[...] = h * n_ref[:N_NODES, 0:1]

    return pl.pallas_call(
        body,
        in_specs=[
            pl.BlockSpec((D, N_NODES), lambda: (0, 0)),
            pl.BlockSpec((D, DP), lambda: (0, 0)),
            pl.BlockSpec((1, DP), lambda: (0, 0)),
            pl.BlockSpec((N_PAD, 2), lambda: (0, 0)),
        ],
        out_specs=[
            pl.BlockSpec((N_NODES, DP), lambda: (0, 0)),
            pl.BlockSpec((N_NODES, DP), lambda: (0, 0)),
        ],
        out_shape=[
            jax.ShapeDtypeStruct((N_NODES, DP), jnp.float32),
            jax.ShapeDtypeStruct((N_NODES, DP), jnp.float32),
        ],
    )(xt, wp, bp, norms)


def _layer_kernel(parts, norms, snorm, h_in, wp, bp, gp, bep):
    def body(p_ref, n_ref, sn_ref, h_ref, w_ref, b_ref, g_ref, be_ref,
             o_ref, os_ref, z_sc, st_sc):
        j = pl.program_id(0)
        i = pl.program_id(1)

        @pl.when(j == 0)
        def _():
            @pl.when(i == 0)
            def _():
                st_sc[...] = jnp.zeros_like(st_sc)

            agg = (p_ref[0] + p_ref[1]) * n_ref[:, 1:2]
            z = jnp.dot(agg, w_ref[...], preferred_element_type=jnp.float32)
            z = (z + b_ref[...]) * sn_ref[...]
            z_sc[pl.ds(i * BLK, BLK), :] = z
            st_sc[0:1, :] += jnp.sum(z, axis=0, keepdims=True)
            st_sc[1:2, :] += jnp.sum(z * z, axis=0, keepdims=True)

        @pl.when(j == 1)
        def _():
            inv_n = 1.0 / N_NODES
            mean = st_sc[0:1, :] * inv_n
            var = st_sc[1:2, :] * inv_n - mean * mean
            zblk = z_sc[pl.ds(i * BLK, BLK), :]
            hn = (zblk - mean) * lax.rsqrt(var + EPS)
            hn = hn * g_ref[...] + be_ref[...]
            h = h_ref[...] + jnp.maximum(hn, 0.0)
            o_ref[...] = h
            os_ref[...] = h * n_ref[:, 0:1]

    return pl.pallas_call(
        body,
        grid=(2, NBLK),
        in_specs=[
            pl.BlockSpec((NC, BLK, DP), lambda j, i: (0, i * (1 - j), 0)),
            pl.BlockSpec((BLK, 2), lambda j, i: (i, 0)),
            pl.BlockSpec((BLK, 1), lambda j, i: (i * (1 - j), 0)),
            pl.BlockSpec((BLK, DP), lambda j, i: (i * j, 0)),
            pl.BlockSpec((DP, DP), lambda j, i: (0, 0)),
            pl.BlockSpec((1, DP), lambda j, i: (0, 0)),
            pl.BlockSpec((1, DP), lambda j, i: (0, 0)),
            pl.BlockSpec((1, DP), lambda j, i: (0, 0)),
        ],
        out_specs=[
            pl.BlockSpec((BLK, DP), lambda j, i: (i * j, 0)),
            pl.BlockSpec((BLK, DP), lambda j, i: (i * j, 0)),
        ],
        out_shape=[
            jax.ShapeDtypeStruct((N_NODES, DP), jnp.float32),
            jax.ShapeDtypeStruct((N_NODES, DP), jnp.float32),
        ],
        scratch_shapes=[
            pltpu.VMEM((N_NODES, DP), jnp.float32),
            pltpu.VMEM((8, DP), jnp.float32),
        ],
    )(parts, norms, snorm, h_in, wp, bp, gp, bep)


def _pool_mlp_kernel(h, gid2d, wr0, br0, wr1, br1, wr2, br2):
    def body(h_ref, g_ref, w0_ref, b0_ref, w1_ref, b1_ref, w2_ref, b2_ref,
             o_ref, acc, cnt):
        i = pl.program_id(0)

        @pl.when(i == 0)
        def _():
            acc[...] = jnp.zeros_like(acc)
            cnt[...] = jnp.zeros_like(cnt)

        giota = lax.broadcasted_iota(jnp.int32, (BLK, N_GRAPHS), 1)
        onehot = (g_ref[...] == giota).astype(jnp.float32)
        acc[...] += lax.dot_general(
            onehot, h_ref[...], (((0,), (0,)), ((), ())),
            preferred_element_type=jnp.float32)
        cnt[...] += lax.dot_general(
            onehot, jnp.ones((BLK, 8), jnp.float32), (((0,), (0,)), ((), ())),
            preferred_element_type=jnp.float32)

        @pl.when(i == NBLK - 1)
        def _():
            hg = acc[...] / jnp.clip(cnt[:, 0:1], 1.0, None)
            y = jnp.dot(hg, w0_ref[...], preferred_element_type=jnp.float32)
            y = jnp.maximum(y + b0_ref[...], 0.0)
            y = jnp.dot(y, w1_ref[...], preferred_element_type=jnp.float32)
            y = jnp.maximum(y + b1_ref[...], 0.0)
            y = jnp.dot(y, w2_ref[...], preferred_element_type=jnp.float32)
            o_ref[...] = y + b2_ref[...]

    return pl.pallas_call(
        body,
        grid=(NBLK,),
        in_specs=[
            pl.BlockSpec((BLK, DP), lambda i: (i, 0)),
            pl.BlockSpec((BLK, 1), lambda i: (i, 0)),
            pl.BlockSpec((DP, 80), lambda i: (0, 0)),
            pl.BlockSpec((1, 80), lambda i: (0, 0)),
            pl.BlockSpec((80, 48), lambda i: (0, 0)),
            pl.BlockSpec((1, 48), lambda i: (0, 0)),
            pl.BlockSpec((48, 128), lambda i: (0, 0)),
            pl.BlockSpec((1, 128), lambda i: (0, 0)),
        ],
        out_specs=pl.BlockSpec((N_GRAPHS, 128), lambda i: (0, 0)),
        out_shape=jax.ShapeDtypeStruct((N_GRAPHS, 128), jnp.float32),
        scratch_shapes=[
            pltpu.VMEM((N_GRAPHS, DP), jnp.float32),
            pltpu.VMEM((N_GRAPHS, 8), jnp.float32),
        ],
    )(h, gid2d, wr0, br0, wr1, br1, wr2, br2)


def _pad2(a, r, c):
    return jnp.pad(a, ((0, r - a.shape[0]), (0, c - a.shape[1])))


def _pad_row(v, c):
    return jnp.pad(v, (0, c - v.shape[0])).reshape(1, c)


def kernel(nodes_feat, edge_index, edges_feat, nodes_num_norm_sqrt,
           edges_num_norm_sqrt, graph_ids,
           W_emb, b_emb, W1, b1, g1, be1, W2, b2, g2, be2,
           W3, b3, g3, be3, W4, b4, g4, be4,
           Wr0, br0, Wr1, br1, Wr2, br2):
    del edges_feat, edges_num_norm_sqrt

    zeros_tile = jnp.zeros((ROWS_PER_TILE, DP), jnp.float32)
    gid2d = graph_ids.reshape(N_NODES, 1)
    src = edge_index[0]
    dst = edge_index[1]

    deg2 = _sc_degrees(src, dst)
    norms = _norms_kernel(deg2)

    h, hs = _embed_kernel(nodes_feat.T, _pad2(W_emb, D, DP),
                          _pad_row(b_emb, DP), norms)

    layer_params = [
        (W1, b1, g1, be1), (W2, b2, g2, be2), (W3, b3, g3, be3), (W4, b4, g4, be4),
    ]
    for (W, b, g, be) in layer_params:
        parts = _sc_agg(hs, src, dst, zeros_tile)
        h, hs = _layer_kernel(parts, norms, nodes_num_norm_sqrt, h,
                              _pad2(W, DP, DP), _pad_row(b, DP),
                              _pad_row(g, DP), _pad_row(be, DP))

    logits = _pool_mlp_kernel(
        h, gid2d,
        _pad2(Wr0, DP, 80), _pad_row(br0, 80),
        _pad2(Wr1, 80, 48), _pad_row(br1, 48),
        _pad2(Wr2, 48, 128), _pad_row(br2, 128),
    )
    return logits[:, :N_CLASSES_OUT]
